# baseline (speedup 1.0000x reference)
import functools

import jax
import jax.numpy as jnp
from jax import lax
from jax.experimental import pallas as pl
from jax.experimental.pallas import tpu as pltpu
from jax.experimental.pallas import tpu_sc as plsc

L = 16
NC = 2
NS = 16
NW = NC * NS
C = 128


def _tc_prep(ns_ref, w_ref, at_ref, h_ref, s_ref, t_ref):
    h = jnp.dot(ns_ref[...], w_ref[...], preferred_element_type=jnp.float32)
    h_ref[...] = h
    st = lax.dot_general(
        at_ref[...], h, (((1,), (1,)), ((), ())),
        preferred_element_type=jnp.float32)
    s_ref[...] = st[0]
    t_ref[...] = st[1]


def _tc_finish(u_ref, as_ref, o_ref):
    u = u_ref[0] + u_ref[1]
    d = (as_ref[0] + as_ref[1])[:, None]
    o_ref[...] = jnp.where(d > 0.0, u / jnp.where(d > 0.0, d, 1.0), 0.0)


def _sc_body(h_hbm, s_hbm, t_hbm, dst_hbm, src_hbm, u_out, a_out,
             dst8, src8, att_b, sd_b, ts_b, hr, cmp_idx0, cmp_idx1,
             cmp_att0, cmp_att1,
             sem_h, sem_sd, sem_ts, sem_u, sem_a, sem_e, u_sh, as_sh,
             *, kj, np_, trash):
    cmp_idx = (cmp_idx0, cmp_idx1)
    cmp_att = (cmp_att0, cmp_att1)
    cid = lax.axis_index("c")
    sid = lax.axis_index("s")
    w = cid * NS + sid
    rows = np_ // NS
    z16 = jnp.zeros((L,), jnp.float32)

    def zw(i, c_):
        for k in range(8):
            hr[0, i, pl.ds(k * L, L)] = z16
        return c_
    lax.fori_loop(0, C, zw, 0)
    for k in range(C // L):
        sd_b[0, pl.ds(k * L, L)] = z16

    def zu(i, c_):
        pltpu.sync_copy(hr.at[0], u_sh.at[pl.ds(sid * rows + i * C, C)])
        pltpu.sync_copy(sd_b.at[0], as_sh.at[pl.ds(sid * rows + i * C, C)])
        return c_
    lax.fori_loop(0, rows // C, zu, 0)

    plsc.subcore_barrier()

    nblk = kj // 8

    def gathers(blk_buf, j, b):
        d_sd = pltpu.async_copy(
            s_hbm.at[dst8.at[blk_buf, j]], sd_b.at[b], sem_sd.at[b])
        d_ts = pltpu.async_copy(
            t_hbm.at[src8.at[blk_buf, j]], ts_b.at[b], sem_ts.at[b])
        d_h = pltpu.async_copy(
            h_hbm.at[src8.at[blk_buf, j]], hr.at[b], sem_h.at[b])
        return d_sd, d_ts, d_h

    pltpu.sync_copy(dst_hbm.at[pl.ds(w * kj, 8)], dst8.at[0])
    pltpu.sync_copy(src_hbm.at[pl.ds(w * kj, 8)], src8.at[0])

    def body(blk, c_):
        cur = lax.rem(blk, 2)
        nxt = lax.rem(blk + 1, 2)
        d_ed = d_es = None
        d_ed = pltpu.async_copy(
            dst_hbm.at[pl.ds(w * kj + lax.min(blk + 1, nblk - 1) * 8, 8)],
            dst8.at[nxt], sem_e.at[0])
        d_es = pltpu.async_copy(
            src_hbm.at[pl.ds(w * kj + lax.min(blk + 1, nblk - 1) * 8, 8)],
            src8.at[nxt], sem_e.at[1])

        trash16 = jnp.full((L,), trash, jnp.int32)
        lane0 = lax.iota(jnp.int32, L) == 0
        zero16 = jnp.zeros((L,), jnp.float32)

        d_g = [None, None]
        d_g[0] = gathers(cur, 0, 0)
        for j in range(8):
            b = j % 2
            if j < 7:
                d_g[1 - b] = gathers(cur, j + 1, 1 - b)
            d_sd, d_ts, d_h = d_g[b]
            d_sd.wait()
            d_ts.wait()

            def att_k(k, c3, b=b):
                x = sd_b[b, pl.ds(k * L, L)] + ts_b[b, pl.ds(k * L, L)]
                x = jnp.maximum(x, 0.2 * x)
                x = jnp.minimum(jnp.maximum(x, -2.0), 2.0)
                att_b[b, pl.ds(k * L, L)] = jnp.exp(x)
                return c3
            lax.fori_loop(0, C // L, att_k, 0)

            d_h.wait()

            for k in range(C // L):
                cmp_idx[b][k] = trash16

            def red16(j16, carry, b=b, j=j):
                cnt, prev, aa, acc = carry
                dv = dst8[cur, j, pl.ds(j16 * L, L)]
                attv = att_b[b, pl.ds(j16 * L, L)]
                for i16 in range(L):
                    i = j16 * L + i16
                    d_i = dv[i16]
                    a_i = attv[i16]
                    flag = d_i != prev
                    cnt = cnt + jnp.where(flag, 1, 0).astype(jnp.int32)
                    new_acc = []
                    for k in range(8):
                        contrib = hr[b, i, pl.ds(k * L, L)] * a_i
                        ak = jnp.where(flag, contrib, acc[k] + contrib)
                        hr[b, cnt, pl.ds(k * L, L)] = ak
                        new_acc.append(ak)
                    acc = tuple(new_acc)
                    aa = jnp.where(flag, a_i, aa + a_i)
                    hi = jnp.broadcast_to(lax.shift_right_logical(cnt, 4),
                                          (L,))
                    lo = jnp.broadcast_to(lax.bitwise_and(cnt, 15), (L,))
                    plsc.store_scatter(
                        cmp_idx[b], [hi, lo],
                        jnp.broadcast_to(d_i, (L,)), mask=lane0)
                    plsc.store_scatter(
                        cmp_att[b], [hi, lo],
                        jnp.broadcast_to(aa, (L,)), mask=lane0)
                    prev = d_i
                return cnt, prev, aa, acc
            cnt_f, _, _, _ = lax.fori_loop(
                0, C // L, red16,
                (jnp.int32(-1), jnp.int32(-1), jnp.float32(0.0),
                 (zero16,) * 8))

            def scat(i, c3, b=b):
                pltpu.sync_copy(hr.at[b, pl.ds(i * L, L)],
                                u_sh.at[cmp_idx[b].at[i]], add=True)
                pltpu.sync_copy(cmp_att[b].at[i],
                                as_sh.at[cmp_idx[b].at[i]], add=True)
                return c3
            lax.fori_loop(0, (cnt_f + L) // L, scat, 0)
        d_ed.wait()
        d_es.wait()
        return c_
    lax.fori_loop(0, nblk, body, 0)

    plsc.subcore_barrier()

    pltpu.sync_copy(u_sh.at[pl.ds(sid * rows, rows)],
                    u_out.at[cid, pl.ds(sid * rows, rows)])
    pltpu.sync_copy(as_sh.at[pl.ds(sid * rows, rows)],
                    a_out.at[cid, pl.ds(sid * rows, rows)])


def kernel(node_states, edges, kernel, kernel_attention):
    n, d = node_states.shape
    u = kernel.shape[1]
    e = edges.shape[0]

    edges = edges.astype(jnp.int32)
    dst = edges[:, 0]
    src = edges[:, 1]

    rb = 512
    np_ = ((n + 1 + rb - 1) // rb) * rb
    kj = ((-(-e // (NW * C)) + 7) // 8) * 8
    ep = NW * kj * C

    ns_p = jnp.pad(node_states, ((0, np_ - n), (0, 0)))
    at = kernel_attention.reshape(2, u)
    dst_p = jnp.concatenate(
        [dst, jnp.full((ep - e,), n, jnp.int32)]).reshape(NW * kj, C)
    src_p = jnp.concatenate(
        [src, jnp.zeros((ep - e,), jnp.int32)]).reshape(NW * kj, C)

    h, s, t = pl.pallas_call(
        _tc_prep,
        grid=(np_ // rb,),
        in_specs=[
            pl.BlockSpec((rb, d), lambda i: (i, 0)),
            pl.BlockSpec((d, u), lambda i: (0, 0)),
            pl.BlockSpec((2, u), lambda i: (0, 0)),
        ],
        out_specs=[
            pl.BlockSpec((rb, u), lambda i: (i, 0)),
            pl.BlockSpec((rb,), lambda i: (i,)),
            pl.BlockSpec((rb,), lambda i: (i,)),
        ],
        out_shape=[
            jax.ShapeDtypeStruct((np_, u), jnp.float32),
            jax.ShapeDtypeStruct((np_,), jnp.float32),
            jax.ShapeDtypeStruct((np_,), jnp.float32),
        ],
    )(ns_p, kernel, at)

    mesh = plsc.VectorSubcoreMesh(core_axis_name="c", subcore_axis_name="s")
    u_part, a_part = pl.kernel(
        functools.partial(_sc_body, kj=kj, np_=np_, trash=n),
        out_type=[
            jax.ShapeDtypeStruct((NC, np_, u), jnp.float32),
            jax.ShapeDtypeStruct((NC, np_), jnp.float32),
        ],
        mesh=mesh,
        compiler_params=pltpu.CompilerParams(needs_layout_passes=False),
        scratch_types=[
            pltpu.VMEM((2, 8, C), jnp.int32),
            pltpu.VMEM((2, 8, C), jnp.int32),
            pltpu.VMEM((2, C), jnp.float32),
            pltpu.VMEM((2, C), jnp.float32),
            pltpu.VMEM((2, C), jnp.float32),
            pltpu.VMEM((2, C, u), jnp.float32),
            pltpu.VMEM((C // L, L), jnp.int32),
            pltpu.VMEM((C // L, L), jnp.int32),
            pltpu.VMEM((C // L, L), jnp.float32),
            pltpu.VMEM((C // L, L), jnp.float32),
            pltpu.SemaphoreType.DMA((2,)),
            pltpu.SemaphoreType.DMA((2,)),
            pltpu.SemaphoreType.DMA((2,)),
            pltpu.SemaphoreType.DMA((2,)),
            pltpu.SemaphoreType.DMA((2,)),
            pltpu.SemaphoreType.DMA((2,)),
            pltpu.VMEM_SHARED((np_, u), jnp.float32),
            pltpu.VMEM_SHARED((np_,), jnp.float32),
        ],
    )(h, s, t, dst_p, src_p)

    out = pl.pallas_call(
        _tc_finish,
        grid=(np_ // rb,),
        in_specs=[
            pl.BlockSpec((NC, rb, u), lambda i: (0, i, 0)),
            pl.BlockSpec((NC, rb), lambda i: (0, i)),
        ],
        out_specs=pl.BlockSpec((rb, u), lambda i: (i, 0)),
        out_shape=jax.ShapeDtypeStruct((np_, u), jnp.float32),
    )(u_part, a_part)

    return out[:n]


# speedup vs baseline: 1.0077x; 1.0077x over previous
import functools

import jax
import jax.numpy as jnp
from jax import lax
from jax.experimental import pallas as pl
from jax.experimental.pallas import tpu as pltpu
from jax.experimental.pallas import tpu_sc as plsc

L = 16
NC = 2
NS = 16
NW = NC * NS
C = 128


def _tc_prep(ns_ref, w_ref, at_ref, h_ref, s_ref, t_ref):
    h = jnp.dot(ns_ref[...], w_ref[...], preferred_element_type=jnp.float32)
    h_ref[...] = h
    st = lax.dot_general(
        at_ref[...], h, (((1,), (1,)), ((), ())),
        preferred_element_type=jnp.float32)
    s_ref[...] = st[0]
    t_ref[...] = st[1]


def _tc_finish(u_ref, as_ref, o_ref):
    u = u_ref[0] + u_ref[1]
    d = (as_ref[0] + as_ref[1])[:, None]
    o_ref[...] = jnp.where(d > 0.0, u / jnp.where(d > 0.0, d, 1.0), 0.0)


def _sc_body(h_hbm, s_hbm, t_hbm, dst_hbm, src_hbm, u_out, a_out,
             dst8, src8, att_b, sd_b, ts_b, hr, cmp_idx0, cmp_idx1,
             sem_h, sem_sd, sem_ts, sem_u, sem_a, sem_e, u_sh, as_sh,
             *, kj, np_, trash):
    cmp_idx = (cmp_idx0, cmp_idx1)
    cid = lax.axis_index("c")
    sid = lax.axis_index("s")
    w = cid * NS + sid
    rows = np_ // NS
    z16 = jnp.zeros((L,), jnp.float32)

    def zw(i, c_):
        for k in range(8):
            hr[0, i, pl.ds(k * L, L)] = z16
        return c_
    lax.fori_loop(0, C, zw, 0)
    for k in range(C // L):
        sd_b[0, pl.ds(k * L, L)] = z16

    def zu(i, c_):
        pltpu.sync_copy(hr.at[0], u_sh.at[pl.ds(sid * rows + i * C, C)])
        pltpu.sync_copy(sd_b.at[0], as_sh.at[pl.ds(sid * rows + i * C, C)])
        return c_
    lax.fori_loop(0, rows // C, zu, 0)

    plsc.subcore_barrier()

    nblk = kj // 8

    def gathers(blk_buf, j, b):
        d_sd = pltpu.async_copy(
            s_hbm.at[dst8.at[blk_buf, j]], sd_b.at[b], sem_sd.at[b])
        d_ts = pltpu.async_copy(
            t_hbm.at[src8.at[blk_buf, j]], ts_b.at[b], sem_ts.at[b])
        d_h = pltpu.async_copy(
            h_hbm.at[src8.at[blk_buf, j]], hr.at[b], sem_h.at[b])
        return d_sd, d_ts, d_h

    pltpu.sync_copy(dst_hbm.at[pl.ds(w * kj, 8)], dst8.at[0])
    pltpu.sync_copy(src_hbm.at[pl.ds(w * kj, 8)], src8.at[0])

    def body(blk, c_):
        cur = lax.rem(blk, 2)
        nxt = lax.rem(blk + 1, 2)
        d_ed = d_es = None
        d_ed = pltpu.async_copy(
            dst_hbm.at[pl.ds(w * kj + lax.min(blk + 1, nblk - 1) * 8, 8)],
            dst8.at[nxt], sem_e.at[0])
        d_es = pltpu.async_copy(
            src_hbm.at[pl.ds(w * kj + lax.min(blk + 1, nblk - 1) * 8, 8)],
            src8.at[nxt], sem_e.at[1])

        trash16 = jnp.full((L,), trash, jnp.int32)
        lane0 = lax.iota(jnp.int32, L) == 0
        zero16 = jnp.zeros((L,), jnp.float32)

        pend = [None, None]
        d_g = [None, None]
        d_g[0] = gathers(cur, 0, 0)
        for j in range(8):
            b = j % 2
            if j < 7:
                if pend[1 - b] is not None:
                    pend[1 - b][0].wait()
                    pend[1 - b][1].wait()
                d_g[1 - b] = gathers(cur, j + 1, 1 - b)
            d_sd, d_ts, d_h = d_g[b]
            d_sd.wait()
            d_ts.wait()

            def att_k(k, c3, b=b):
                x = sd_b[b, pl.ds(k * L, L)] + ts_b[b, pl.ds(k * L, L)]
                x = jnp.maximum(x, 0.2 * x)
                x = jnp.minimum(jnp.maximum(x, -2.0), 2.0)
                att_b[b, pl.ds(k * L, L)] = jnp.exp(x)
                return c3
            lax.fori_loop(0, C // L, att_k, 0)
            d_sa = pltpu.async_copy(
                att_b.at[b], as_sh.at[dst8.at[cur, j]], sem_a.at[b],
                add=True)

            d_h.wait()

            for k in range(C // 32):
                cmp_idx[b][k, pl.ds(0, L)] = trash16
                cmp_idx[b][k, pl.ds(L, L)] = trash16

            def red16(j16, carry, b=b, j=j):
                cnt, prev, acc = carry
                dv = dst8[cur, j, pl.ds(j16 * L, L)]
                attv = att_b[b, pl.ds(j16 * L, L)]
                for i16 in range(L):
                    i = j16 * L + i16
                    d_i = dv[i16]
                    a_i = attv[i16]
                    flag = d_i != prev
                    cnt = cnt + jnp.where(flag, 1, 0).astype(jnp.int32)
                    new_acc = []
                    for k in range(8):
                        contrib = hr[b, i, pl.ds(k * L, L)] * a_i
                        ak = jnp.where(flag, contrib, acc[k] + contrib)
                        hr[b, cnt, pl.ds(k * L, L)] = ak
                        new_acc.append(ak)
                    acc = tuple(new_acc)
                    hi = jnp.broadcast_to(lax.shift_right_logical(cnt, 5),
                                          (L,))
                    lo = jnp.broadcast_to(lax.bitwise_and(cnt, 31), (L,))
                    plsc.store_scatter(
                        cmp_idx[b], [hi, lo],
                        jnp.broadcast_to(d_i, (L,)),
                        mask=jnp.logical_and(lane0, flag))
                    prev = d_i
                return cnt, prev, acc
            cnt_f, _, _ = lax.fori_loop(
                0, C // L, red16,
                (jnp.int32(-1), jnp.int32(-1), (zero16,) * 8))

            d_su = pltpu.async_copy(
                hr.at[b, pl.ds(0, 32)], u_sh.at[cmp_idx[b].at[0]],
                sem_u.at[b], add=True)
            pend[b] = (d_su, d_sa)

            def scat(g, c3, b=b):
                pltpu.sync_copy(hr.at[b, pl.ds(g * 32, 32)],
                                u_sh.at[cmp_idx[b].at[g]], add=True)
                return c3
            lax.fori_loop(1, (cnt_f + 32) // 32, scat, 0)
        for b in range(2):
            pend[b][0].wait()
            pend[b][1].wait()
        d_ed.wait()
        d_es.wait()
        return c_
    lax.fori_loop(0, nblk, body, 0)

    plsc.subcore_barrier()

    pltpu.sync_copy(u_sh.at[pl.ds(sid * rows, rows)],
                    u_out.at[cid, pl.ds(sid * rows, rows)])
    pltpu.sync_copy(as_sh.at[pl.ds(sid * rows, rows)],
                    a_out.at[cid, pl.ds(sid * rows, rows)])


def kernel(node_states, edges, kernel, kernel_attention):
    n, d = node_states.shape
    u = kernel.shape[1]
    e = edges.shape[0]

    edges = edges.astype(jnp.int32)
    dst = edges[:, 0]
    src = edges[:, 1]

    rb = 512
    np_ = ((n + 1 + rb - 1) // rb) * rb
    kj = ((-(-e // (NW * C)) + 7) // 8) * 8
    ep = NW * kj * C

    ns_p = jnp.pad(node_states, ((0, np_ - n), (0, 0)))
    at = kernel_attention.reshape(2, u)
    dst_p = jnp.concatenate(
        [dst, jnp.full((ep - e,), n, jnp.int32)]).reshape(NW * kj, C)
    src_p = jnp.concatenate(
        [src, jnp.zeros((ep - e,), jnp.int32)]).reshape(NW * kj, C)

    h, s, t = pl.pallas_call(
        _tc_prep,
        grid=(np_ // rb,),
        in_specs=[
            pl.BlockSpec((rb, d), lambda i: (i, 0)),
            pl.BlockSpec((d, u), lambda i: (0, 0)),
            pl.BlockSpec((2, u), lambda i: (0, 0)),
        ],
        out_specs=[
            pl.BlockSpec((rb, u), lambda i: (i, 0)),
            pl.BlockSpec((rb,), lambda i: (i,)),
            pl.BlockSpec((rb,), lambda i: (i,)),
        ],
        out_shape=[
            jax.ShapeDtypeStruct((np_, u), jnp.float32),
            jax.ShapeDtypeStruct((np_,), jnp.float32),
            jax.ShapeDtypeStruct((np_,), jnp.float32),
        ],
    )(ns_p, kernel, at)

    mesh = plsc.VectorSubcoreMesh(core_axis_name="c", subcore_axis_name="s")
    u_part, a_part = pl.kernel(
        functools.partial(_sc_body, kj=kj, np_=np_, trash=n),
        out_type=[
            jax.ShapeDtypeStruct((NC, np_, u), jnp.float32),
            jax.ShapeDtypeStruct((NC, np_), jnp.float32),
        ],
        mesh=mesh,
        compiler_params=pltpu.CompilerParams(needs_layout_passes=False),
        scratch_types=[
            pltpu.VMEM((2, 8, C), jnp.int32),
            pltpu.VMEM((2, 8, C), jnp.int32),
            pltpu.VMEM((2, C), jnp.float32),
            pltpu.VMEM((2, C), jnp.float32),
            pltpu.VMEM((2, C), jnp.float32),
            pltpu.VMEM((2, C, u), jnp.float32),
            pltpu.VMEM((C // 32, 32), jnp.int32),
            pltpu.VMEM((C // 32, 32), jnp.int32),
            pltpu.SemaphoreType.DMA((2,)),
            pltpu.SemaphoreType.DMA((2,)),
            pltpu.SemaphoreType.DMA((2,)),
            pltpu.SemaphoreType.DMA((2,)),
            pltpu.SemaphoreType.DMA((2,)),
            pltpu.SemaphoreType.DMA((2,)),
            pltpu.VMEM_SHARED((np_, u), jnp.float32),
            pltpu.VMEM_SHARED((np_,), jnp.float32),
        ],
    )(h, s, t, dst_p, src_p)

    out = pl.pallas_call(
        _tc_finish,
        grid=(np_ // rb,),
        in_specs=[
            pl.BlockSpec((NC, rb, u), lambda i: (0, i, 0)),
            pl.BlockSpec((NC, rb), lambda i: (0, i)),
        ],
        out_specs=pl.BlockSpec((rb, u), lambda i: (i, 0)),
        out_shape=jax.ShapeDtypeStruct((np_, u), jnp.float32),
    )(u_part, a_part)

    return out[:n]


# speedup vs baseline: 1.2840x; 1.2742x over previous
import functools

import jax
import jax.numpy as jnp
from jax import lax
from jax.experimental import pallas as pl
from jax.experimental.pallas import tpu as pltpu
from jax.experimental.pallas import tpu_sc as plsc

L = 16
NC = 2
NS = 16
NW = NC * NS
C = 128
C2 = 64
BLK = 16
DEP = 4


def _tc_prep(ns_ref, w_ref, at_ref, h_ref, s_ref, t_ref):
    h = jnp.dot(ns_ref[...], w_ref[...], preferred_element_type=jnp.float32)
    h_ref[...] = h
    st = lax.dot_general(
        at_ref[...], h, (((1,), (1,)), ((), ())),
        preferred_element_type=jnp.float32)
    s_ref[...] = st[0]
    t_ref[...] = st[1]


def _tc_finish(u_ref, as_ref, o_ref):
    u = u_ref[0] + u_ref[1]
    d = (as_ref[0] + as_ref[1])[:, None]
    o_ref[...] = jnp.where(d > 0.0, u / jnp.where(d > 0.0, d, 1.0), 0.0)


def _sc_body(h_hbm, s_hbm, t_hbm, dst_hbm, src_hbm, u_out, a_out,
             dst8, src8, att_b, sd_b, ts_b, hr,
             sem_h, sem_sd, sem_ts, sem_u, sem_a, sem_e, u_sh, as_sh,
             *, kj, np_):
    cid = lax.axis_index("c")
    sid = lax.axis_index("s")
    w = cid * NS + sid
    rows = np_ // NS
    z16 = jnp.zeros((L,), jnp.float32)

    def zw(i, c_):
        for k in range(8):
            hr[0, i, pl.ds(k * L, L)] = z16
        return c_
    lax.fori_loop(0, C2, zw, 0)
    for k in range(C2 // L):
        sd_b[0, pl.ds(k * L, L)] = z16

    def zu(i, c_):
        pltpu.sync_copy(hr.at[0], u_sh.at[pl.ds(sid * rows + i * C2, C2)])
        pltpu.sync_copy(sd_b.at[0], as_sh.at[pl.ds(sid * rows + i * C2, C2)])
        return c_
    lax.fori_loop(0, rows // C2, zu, 0)

    plsc.subcore_barrier()

    nblk = kj // BLK

    def gathers(blk_buf, j, b):
        d_sd = pltpu.async_copy(
            s_hbm.at[dst8.at[blk_buf, j]], sd_b.at[b], sem_sd.at[b])
        d_ts = pltpu.async_copy(
            t_hbm.at[src8.at[blk_buf, j]], ts_b.at[b], sem_ts.at[b])
        d_h = pltpu.async_copy(
            h_hbm.at[src8.at[blk_buf, j]], hr.at[b], sem_h.at[b])
        return d_sd, d_ts, d_h

    pltpu.sync_copy(dst_hbm.at[pl.ds(w * kj, BLK)], dst8.at[0])
    pltpu.sync_copy(src_hbm.at[pl.ds(w * kj, BLK)], src8.at[0])

    def body(blk, c_):
        cur = lax.rem(blk, 2)
        nxt = lax.rem(blk + 1, 2)
        nb = lax.min(blk + 1, nblk - 1) * BLK
        d_ed = pltpu.async_copy(
            dst_hbm.at[pl.ds(w * kj + nb, BLK)], dst8.at[nxt], sem_e.at[0])
        d_es = pltpu.async_copy(
            src_hbm.at[pl.ds(w * kj + nb, BLK)], src8.at[nxt], sem_e.at[1])

        pend = [None] * DEP
        d_g = [None] * DEP
        for j in range(DEP - 1):
            d_g[j] = gathers(cur, j, j)
        for j in range(BLK):
            b = j % DEP
            jn = j + DEP - 1
            if jn < BLK:
                bn = jn % DEP
                if pend[bn] is not None:
                    pend[bn][0].wait()
                    pend[bn][1].wait()
                    pend[bn] = None
                d_g[bn] = gathers(cur, jn, bn)
            d_sd, d_ts, d_h = d_g[b]
            d_sd.wait()
            d_ts.wait()

            def att_k(k, c3, b=b):
                x = sd_b[b, pl.ds(k * L, L)] + ts_b[b, pl.ds(k * L, L)]
                x = jnp.maximum(x, 0.2 * x)
                x = jnp.minimum(jnp.maximum(x, -2.0), 2.0)
                att_b[b, pl.ds(k * L, L)] = jnp.exp(x)
                return c3
            lax.fori_loop(0, C2 // L, att_k, 0)
            d_sa = pltpu.async_copy(
                att_b.at[b], as_sh.at[dst8.at[cur, j]], sem_a.at[b],
                add=True)

            d_h.wait()

            def wbody(j16, c3, b=b):
                attv = att_b[b, pl.ds(j16 * L, L)]
                for i16 in range(L):
                    a = attv[i16]
                    i = j16 * L + i16
                    for k in range(8):
                        hr[b, i, pl.ds(k * L, L)] = (
                            hr[b, i, pl.ds(k * L, L)] * a)
                return c3
            lax.fori_loop(0, C2 // L, wbody, 0)
            d_su = pltpu.async_copy(
                hr.at[b], u_sh.at[dst8.at[cur, j]], sem_u.at[b], add=True)
            pend[b] = (d_su, d_sa)
        for b in range(DEP):
            if pend[b] is not None:
                pend[b][0].wait()
                pend[b][1].wait()
        d_ed.wait()
        d_es.wait()
        return c_
    lax.fori_loop(0, nblk, body, 0)

    plsc.subcore_barrier()

    pltpu.sync_copy(u_sh.at[pl.ds(sid * rows, rows)],
                    u_out.at[cid, pl.ds(sid * rows, rows)])
    pltpu.sync_copy(as_sh.at[pl.ds(sid * rows, rows)],
                    a_out.at[cid, pl.ds(sid * rows, rows)])


def kernel(node_states, edges, kernel, kernel_attention):
    n, d = node_states.shape
    u = kernel.shape[1]
    e = edges.shape[0]

    edges = edges.astype(jnp.int32)
    dst = edges[:, 0]
    src = edges[:, 1]

    rb = 512
    np_ = ((n + 1 + rb - 1) // rb) * rb
    kj = ((-(-e // (NW * C2)) + BLK - 1) // BLK) * BLK
    ep = NW * kj * C2

    ns_p = jnp.pad(node_states, ((0, np_ - n), (0, 0)))
    at = kernel_attention.reshape(2, u)
    dst_p = jnp.concatenate(
        [dst, jnp.full((ep - e,), n, jnp.int32)]).reshape(NW * kj, C2)
    src_p = jnp.concatenate(
        [src, jnp.zeros((ep - e,), jnp.int32)]).reshape(NW * kj, C2)

    h, s, t = pl.pallas_call(
        _tc_prep,
        grid=(np_ // rb,),
        in_specs=[
            pl.BlockSpec((rb, d), lambda i: (i, 0)),
            pl.BlockSpec((d, u), lambda i: (0, 0)),
            pl.BlockSpec((2, u), lambda i: (0, 0)),
        ],
        out_specs=[
            pl.BlockSpec((rb, u), lambda i: (i, 0)),
            pl.BlockSpec((rb,), lambda i: (i,)),
            pl.BlockSpec((rb,), lambda i: (i,)),
        ],
        out_shape=[
            jax.ShapeDtypeStruct((np_, u), jnp.float32),
            jax.ShapeDtypeStruct((np_,), jnp.float32),
            jax.ShapeDtypeStruct((np_,), jnp.float32),
        ],
    )(ns_p, kernel, at)

    mesh = plsc.VectorSubcoreMesh(core_axis_name="c", subcore_axis_name="s")
    u_part, a_part = pl.kernel(
        functools.partial(_sc_body, kj=kj, np_=np_),
        out_type=[
            jax.ShapeDtypeStruct((NC, np_, u), jnp.float32),
            jax.ShapeDtypeStruct((NC, np_), jnp.float32),
        ],
        mesh=mesh,
        compiler_params=pltpu.CompilerParams(needs_layout_passes=False),
        scratch_types=[
            pltpu.VMEM((2, BLK, C2), jnp.int32),
            pltpu.VMEM((2, BLK, C2), jnp.int32),
            pltpu.VMEM((DEP, C2), jnp.float32),
            pltpu.VMEM((DEP, C2), jnp.float32),
            pltpu.VMEM((DEP, C2), jnp.float32),
            pltpu.VMEM((DEP, C2, u), jnp.float32),
            pltpu.SemaphoreType.DMA((DEP,)),
            pltpu.SemaphoreType.DMA((DEP,)),
            pltpu.SemaphoreType.DMA((DEP,)),
            pltpu.SemaphoreType.DMA((DEP,)),
            pltpu.SemaphoreType.DMA((DEP,)),
            pltpu.SemaphoreType.DMA((2,)),
            pltpu.VMEM_SHARED((np_, u), jnp.float32),
            pltpu.VMEM_SHARED((np_,), jnp.float32),
        ],
    )(h, s, t, dst_p, src_p)

    out = pl.pallas_call(
        _tc_finish,
        grid=(np_ // rb,),
        in_specs=[
            pl.BlockSpec((NC, rb, u), lambda i: (0, i, 0)),
            pl.BlockSpec((NC, rb), lambda i: (0, i)),
        ],
        out_specs=pl.BlockSpec((rb, u), lambda i: (i, 0)),
        out_shape=jax.ShapeDtypeStruct((np_, u), jnp.float32),
    )(u_part, a_part)

    return out[:n]
